# baseline (speedup 1.0000x reference)
import functools

import jax
import jax.numpy as jnp
from jax.experimental import pallas as pl
from jax.experimental.pallas import tpu as pltpu

_SCALING = 8.0 / 64.0

_BM = 512
_NCHUNK = 16
_NIN = 3
_NOUT = 3


def _moelora_pipeline(x_hbm, tw_ref, wa_ref, wb_ref, o_hbm,
                      xbuf, obuf, insem, outsem):
    def in_copy(i):
        return pltpu.make_async_copy(
            x_hbm.at[pl.ds(i * _BM, _BM), :],
            xbuf.at[i % _NIN],
            insem.at[i % _NIN],
        )

    def out_copy(i):
        return pltpu.make_async_copy(
            obuf.at[i % _NOUT],
            o_hbm.at[pl.ds(i * _BM, _BM), :],
            outsem.at[i % _NOUT],
        )

    for i in range(_NIN):
        in_copy(i).start()

    for i in range(_NCHUNK):
        in_copy(i).wait()
        h = jax.lax.dot_general(
            xbuf[i % _NIN], wa_ref[...],
            dimension_numbers=(((1,), (1,)), ((), ())),
            preferred_element_type=jnp.float32,
        )
        tw = tw_ref[pl.ds(i * _BM, _BM), :]
        h = h * jnp.where(tw != 0.0, tw * _SCALING, jnp.zeros((), jnp.float32))
        if i >= _NOUT:
            out_copy(i - _NOUT).wait()
        obuf[i % _NOUT] = jax.lax.dot_general(
            h.astype(jnp.bfloat16), wb_ref[...],
            dimension_numbers=(((1,), (1,)), ((), ())),
            preferred_element_type=jnp.float32,
        )
        out_copy(i).start()
        if i + _NIN < _NCHUNK:
            in_copy(i + _NIN).start()

    for i in range(_NCHUNK - _NOUT, _NCHUNK):
        out_copy(i).wait()


@functools.partial(jax.jit, static_argnames=())
def kernel(x, type_weight, W_A, W_B):
    B, S, D_IN = x.shape
    D_OUT, R = W_B.shape
    M = B * S
    x2 = x.reshape(M, D_IN)
    tw2 = type_weight.reshape(M, 1)
    wb16 = W_B.astype(jnp.bfloat16)

    out = pl.pallas_call(
        _moelora_pipeline,
        in_specs=[
            pl.BlockSpec(memory_space=pltpu.MemorySpace.HBM),
            pl.BlockSpec(memory_space=pltpu.MemorySpace.VMEM),
            pl.BlockSpec(memory_space=pltpu.MemorySpace.VMEM),
            pl.BlockSpec(memory_space=pltpu.MemorySpace.VMEM),
        ],
        out_specs=pl.BlockSpec(memory_space=pltpu.MemorySpace.HBM),
        out_shape=jax.ShapeDtypeStruct((M, D_OUT), x.dtype),
        scratch_shapes=[
            pltpu.VMEM((_NIN, _BM, D_IN), jnp.float32),
            pltpu.VMEM((_NOUT, _BM, D_OUT), jnp.float32),
            pltpu.SemaphoreType.DMA((_NIN,)),
            pltpu.SemaphoreType.DMA((_NOUT,)),
        ],
    )(x2, tw2, W_A, wb16)
    return out.reshape(B, S, D_OUT)


# speedup vs baseline: 1.0065x; 1.0065x over previous
import functools

import jax
import jax.numpy as jnp
from jax.experimental import pallas as pl
from jax.experimental.pallas import tpu as pltpu

_SCALING = 8.0 / 64.0

_BM = 512
_SIZES = (128, 128, 256) + (512,) * 14 + (256, 128, 128)
_OFFS = tuple(sum(_SIZES[:i]) for i in range(len(_SIZES)))
_NCHUNK = len(_SIZES)
_NIN = 3
_NOUT = 3


def _moelora_pipeline(x_hbm, tw_ref, wa_ref, wb_ref, o_hbm,
                      xbuf, obuf, insem, outsem):
    def in_copy(i):
        return pltpu.make_async_copy(
            x_hbm.at[pl.ds(_OFFS[i], _SIZES[i]), :],
            xbuf.at[i % _NIN, pl.ds(0, _SIZES[i]), :],
            insem.at[i % _NIN],
        )

    def out_copy(i):
        return pltpu.make_async_copy(
            obuf.at[i % _NOUT, pl.ds(0, _SIZES[i]), :],
            o_hbm.at[pl.ds(_OFFS[i], _SIZES[i]), :],
            outsem.at[i % _NOUT],
        )

    for i in range(_NIN):
        in_copy(i).start()

    for i in range(_NCHUNK):
        in_copy(i).wait()
        h = jax.lax.dot_general(
            xbuf[i % _NIN, pl.ds(0, _SIZES[i]), :], wa_ref[...],
            dimension_numbers=(((1,), (1,)), ((), ())),
            preferred_element_type=jnp.float32,
        )
        tw = tw_ref[pl.ds(_OFFS[i], _SIZES[i]), :]
        h = h * jnp.where(tw != 0.0, tw * _SCALING, jnp.zeros((), jnp.float32))
        if i >= _NOUT:
            out_copy(i - _NOUT).wait()
        obuf[i % _NOUT, pl.ds(0, _SIZES[i]), :] = jax.lax.dot_general(
            h.astype(jnp.bfloat16), wb_ref[...],
            dimension_numbers=(((1,), (1,)), ((), ())),
            preferred_element_type=jnp.float32,
        )
        out_copy(i).start()
        if i + _NIN < _NCHUNK:
            in_copy(i + _NIN).start()

    for i in range(_NCHUNK - _NOUT, _NCHUNK):
        out_copy(i).wait()


@functools.partial(jax.jit, static_argnames=())
def kernel(x, type_weight, W_A, W_B):
    B, S, D_IN = x.shape
    D_OUT, R = W_B.shape
    M = B * S
    x2 = x.reshape(M, D_IN)
    tw2 = type_weight.reshape(M, 1)
    wb16 = W_B.astype(jnp.bfloat16)

    out = pl.pallas_call(
        _moelora_pipeline,
        in_specs=[
            pl.BlockSpec(memory_space=pltpu.MemorySpace.HBM),
            pl.BlockSpec(memory_space=pltpu.MemorySpace.VMEM),
            pl.BlockSpec(memory_space=pltpu.MemorySpace.VMEM),
            pl.BlockSpec(memory_space=pltpu.MemorySpace.VMEM),
        ],
        out_specs=pl.BlockSpec(memory_space=pltpu.MemorySpace.HBM),
        out_shape=jax.ShapeDtypeStruct((M, D_OUT), x.dtype),
        scratch_shapes=[
            pltpu.VMEM((_NIN, _BM, D_IN), jnp.float32),
            pltpu.VMEM((_NOUT, _BM, D_OUT), jnp.float32),
            pltpu.SemaphoreType.DMA((_NIN,)),
            pltpu.SemaphoreType.DMA((_NOUT,)),
        ],
    )(x2, tw2, W_A, wb16)
    return out.reshape(B, S, D_OUT)
